# baseline (speedup 1.0000x reference)
import math

import jax
import jax.numpy as jnp
from jax import lax
from jax.experimental import pallas as pl
from jax.experimental.pallas import tpu as pltpu
from jax.experimental.pallas import tpu_sc as plsc

VOCAB = 100000
EMBED_DIM = 128
BATCH = 4
SEQ_LEN = 4096
TOTAL = BATCH * SEQ_LEN
SCALE = math.sqrt(EMBED_DIM)

_info = plsc.get_sparse_core_info()
NC, NS = _info.num_cores, _info.num_subcores
NW = NC * NS
CHUNK = SEQ_LEN // NW
LANES = EMBED_DIM // 16


def _body(x_hbm, tok_hbm, pos_hbm, out_hbm,
          idx_v, pos_v, tok0_v, tok1_v, tok2_v, tok3_v, sh_v,
          isem, psem, gsem0, gsem1, gsem2, gsem3, osem0, osem1, osem2, osem3,
          csem0, csem1, csem2, csem3):
    sid = lax.axis_index("s")
    wid = sid * NC + lax.axis_index("c")
    tok_bufs = (tok0_v, tok1_v, tok2_v, tok3_v)
    gsems = (gsem0, gsem1, gsem2, gsem3)
    osems = (osem0, osem1, osem2, osem3)
    csems = (csem0, csem1, csem2, csem3)

    pcp = pltpu.async_copy(pos_hbm.at[pl.ds(wid * CHUNK, CHUNK)], pos_v, psem)
    icp = pltpu.async_copy(x_hbm.at[:, pl.ds(wid * CHUNK, CHUNK)], idx_v, isem)
    icp.wait()

    gcps = [pltpu.async_copy(tok_hbm.at[idx_v.at[c]], tok_bufs[c], gsems[c])
            for c in range(BATCH)]

    pcp.wait()
    ocps = []
    for c in range(BATCH):
        gcps[c].wait()
        tok_v = tok_bufs[c]

        def row_body(r, _):
            for cc in range(LANES):
                sl = pl.ds(cc * 16, 16)
                tok_v[r, sl] = tok_v[r, sl] * SCALE + pos_v[r, sl]
            return 0

        lax.fori_loop(0, CHUNK, row_body, 0)
        if c < 2:
            pltpu.async_copy(tok_v, sh_v.at[sid, c], csems[c]).wait()
            ocps.append(pltpu.async_copy(
                sh_v.at[sid, c],
                out_hbm.at[pl.ds(c * SEQ_LEN + wid * CHUNK, CHUNK)],
                osems[c]))
        else:
            ocps.append(pltpu.async_copy(
                tok_v, out_hbm.at[pl.ds(c * SEQ_LEN + wid * CHUNK, CHUNK)],
                osems[c]))
    for ocp in ocps:
        ocp.wait()


@jax.jit
def kernel(x, token_table, pos_table):
    mesh = plsc.VectorSubcoreMesh(core_axis_name="c", subcore_axis_name="s")
    run = pl.kernel(
        _body,
        out_type=jax.ShapeDtypeStruct((TOTAL, EMBED_DIM), jnp.float32),
        mesh=mesh,
        scratch_types=[
            pltpu.VMEM((BATCH, CHUNK), jnp.int32),
            pltpu.VMEM((CHUNK, EMBED_DIM), jnp.float32),
            pltpu.VMEM((CHUNK, EMBED_DIM), jnp.float32),
            pltpu.VMEM((CHUNK, EMBED_DIM), jnp.float32),
            pltpu.VMEM((CHUNK, EMBED_DIM), jnp.float32),
            pltpu.VMEM((CHUNK, EMBED_DIM), jnp.float32),
            pltpu.VMEM_SHARED((NS, 2, CHUNK, EMBED_DIM), jnp.float32),
        ] + [pltpu.SemaphoreType.DMA] * 14,
    )
    out = run(x.astype(jnp.int32), token_table, pos_table)
    return out.reshape(BATCH, SEQ_LEN, EMBED_DIM)


# speedup vs baseline: 1.0549x; 1.0549x over previous
import math

import jax
import jax.numpy as jnp
from jax import lax
from jax.experimental import pallas as pl
from jax.experimental.pallas import tpu as pltpu
from jax.experimental.pallas import tpu_sc as plsc

VOCAB = 100000
EMBED_DIM = 128
BATCH = 4
SEQ_LEN = 4096
TOTAL = BATCH * SEQ_LEN
SCALE = math.sqrt(EMBED_DIM)

_info = plsc.get_sparse_core_info()
NC, NS = _info.num_cores, _info.num_subcores
NW = NC * NS
CHUNK = SEQ_LEN // NW
LANES = EMBED_DIM // 16


def _body(x_hbm, tok_hbm, pos_hbm, out_hbm,
          idx_v, pos_v, tok0_v, tok1_v, tok2_v, tok3_v,
          isem, psem, gsem0, gsem1, gsem2, gsem3, osem0, osem1, osem2, osem3):
    wid = lax.axis_index("s") * NC + lax.axis_index("c")
    tok_bufs = (tok0_v, tok1_v, tok2_v, tok3_v)
    gsems = (gsem0, gsem1, gsem2, gsem3)
    osems = (osem0, osem1, osem2, osem3)

    pcp = pltpu.async_copy(pos_hbm.at[pl.ds(wid * CHUNK, CHUNK)], pos_v, psem)
    icp = pltpu.async_copy(x_hbm.at[:, pl.ds(wid * CHUNK, CHUNK)], idx_v, isem)
    icp.wait()

    gcps = [pltpu.async_copy(tok_hbm.at[idx_v.at[c]], tok_bufs[c], gsems[c])
            for c in range(BATCH)]

    pcp.wait()
    ocps = []
    for c in range(BATCH):
        gcps[c].wait()
        tok_v = tok_bufs[c]

        def row_body(r, _):
            for cc in range(LANES):
                sl = pl.ds(cc * 16, 16)
                tok_v[r, sl] = tok_v[r, sl] * SCALE + pos_v[r, sl]
            return 0

        lax.fori_loop(0, CHUNK, row_body, 0)
        ocps.append(pltpu.async_copy(
            tok_v, out_hbm.at[pl.ds(c * SEQ_LEN + wid * CHUNK, CHUNK)],
            osems[c]))
    for ocp in ocps:
        ocp.wait()


@jax.jit
def kernel(x, token_table, pos_table):
    mesh = plsc.VectorSubcoreMesh(core_axis_name="c", subcore_axis_name="s")
    run = pl.kernel(
        _body,
        out_type=jax.ShapeDtypeStruct((TOTAL, EMBED_DIM), jnp.float32),
        mesh=mesh,
        scratch_types=[
            pltpu.VMEM((BATCH, CHUNK), jnp.int32),
            pltpu.VMEM((CHUNK, EMBED_DIM), jnp.float32),
            pltpu.VMEM((CHUNK, EMBED_DIM), jnp.float32),
            pltpu.VMEM((CHUNK, EMBED_DIM), jnp.float32),
            pltpu.VMEM((CHUNK, EMBED_DIM), jnp.float32),
            pltpu.VMEM((CHUNK, EMBED_DIM), jnp.float32),
        ] + [pltpu.SemaphoreType.DMA] * 10,
    )
    out = run(x.astype(jnp.int32), token_table, pos_table)
    return out.reshape(BATCH, SEQ_LEN, EMBED_DIM)
